# baseline (speedup 1.0000x reference)
import functools
import math

import jax
import jax.numpy as jnp
import numpy as np
from jax import lax
from jax.experimental import pallas as pl
from jax.experimental.pallas import tpu as pltpu
from jax.experimental.pallas import tpu_sc as plsc

B = 2
N = 35344
D = 256
NH = 8
NL = 1
DFF = 512
KS = 5
NP_ = KS * KS
KEEP = 0.2
HGRID = 188
WGRID = 188
HSIZE = 188.0
DH = D // NH
FG = math.ceil(N * KEEP)
QN = math.ceil(FG * KEEP)

LQP = 7168
NPTS = 4 * NP_
PG = HGRID + 2
TOTQ = B * LQP * NH
NTILES = 32
QPT = TOTQ // NTILES
CQ = 16
NCHUNK = QPT // CQ


def _kern_pts():
    start = -(KS - 1) / 2
    end = (KS - 1) / 2
    idx = np.linspace(start, end, KS)
    i, j = np.meshgrid(idx, idx, indexing='ij')
    kern = np.stack([j, i], axis=-1).reshape(-1, 2) / KS
    return kern.astype(np.float32)


def _kxy(shape_like):
    pi = jax.lax.broadcasted_iota(jnp.int32, (1, NP_), 1)
    kx = ((pi % KS) - (KS - 1) // 2).astype(jnp.float32) / KS
    ky = ((pi // KS) - (KS - 1) // 2).astype(jnp.float32) / KS
    return kx, ky


def _ln(x, g, b, eps=1e-5):
    mu = x.mean(-1, keepdims=True)
    var = ((x - mu) ** 2).mean(-1, keepdims=True)
    return (x - mu) / jnp.sqrt(var + eps) * g + b


def _mha_small(q, k, v, p):
    wq, wk, wv = jnp.split(p['in_proj_w'], 3, axis=0)
    bq, bk, bv = jnp.split(p['in_proj_b'], 3, axis=0)
    Bq, L, _ = q.shape

    def proj(x, w, bb):
        return (x @ w.T + bb).reshape(Bq, -1, NH, DH).transpose(0, 2, 1, 3)

    qh = proj(q, wq, bq)
    kh = proj(k, wk, bk)
    vh = proj(v, wv, bv)
    attn = jax.nn.softmax(qh @ kh.transpose(0, 1, 3, 2) / np.sqrt(DH), axis=-1)
    out = (attn @ vh).transpose(0, 2, 1, 3).reshape(Bq, L, D)
    return out @ p['mha_out_w'].T + p['mha_out_b']


def _gat(x, idx):
    ib = jnp.broadcast_to(idx[..., None], idx.shape + (x.shape[-1],))
    return jnp.take_along_axis(x, ib, axis=1)


NPADT = 35456


def _thresh_body(s_ref, o_ref):
    x = s_ref[0]
    k = jax.lax.bitcast_convert_type(x, jnp.uint32)
    neg = (k >> 31).astype(jnp.uint32)
    key = jnp.where(neg == 1, ~k, k | jnp.uint32(0x80000000))

    def find(kth):
        def bit(i, t):
            cand = t | (jnp.uint32(1) << (jnp.uint32(30) - i.astype(jnp.uint32)))
            c = jnp.sum((key >= cand).astype(jnp.int32))
            return jnp.where(c >= kth, cand, t)
        def bit31(t):
            cand = t | jnp.uint32(0x80000000)
            c = jnp.sum((key >= cand).astype(jnp.int32))
            return jnp.where(c >= kth, cand, t)
        t = bit31(jnp.uint32(0))
        return lax.fori_loop(0, 31, bit, t)

    tq = find(QN)
    tf = find(FG)
    lane = jax.lax.broadcasted_iota(jnp.int32, (1, 1, 128), 2)
    o_ref[...] = jnp.where(lane == 1, jnp.full((1, 1, 128), tf),
                           jnp.full((1, 1, 128), tq))


def _thresh(scorep):
    out = pl.pallas_call(
        _thresh_body,
        out_shape=jax.ShapeDtypeStruct((B, 1, 128), jnp.uint32),
        grid=(B,),
        in_specs=[pl.BlockSpec((1, NPADT // 128, 128), lambda b: (b, 0, 0))],
        out_specs=pl.BlockSpec((1, 1, 128), lambda b: (b, 0, 0)),
    )(scorep)
    return out.reshape(B, 128)


def _select_indices(score_mask):
    scorep = jnp.pad(score_mask, ((0, 0), (0, NPADT - N)),
                     constant_values=-np.inf).reshape(B, NPADT // 128, 128)
    thr = _thresh(scorep)
    tq, tf = thr[:, 0:1], thr[:, 1:2]
    k = jax.lax.bitcast_convert_type(score_mask, jnp.uint32)
    neg = (k >> 31).astype(jnp.uint32)
    key = jnp.where(neg == 1, ~k, k | jnp.uint32(0x80000000))
    gtA = key > tq
    eqA = key == tq
    fillA = QN - jnp.sum(gtA, 1, keepdims=True)
    mA = gtA | (eqA & (jnp.cumsum(eqA, 1) <= fillA))
    gtS = key > tf
    eqS = key == tf
    fillS = FG - jnp.sum(gtS, 1, keepdims=True)
    mS = gtS | (eqS & (jnp.cumsum(eqS, 1) <= fillS))
    mB = mS & ~mA
    posA = jnp.cumsum(mA.astype(jnp.int32), 1) - 1
    posB = QN + jnp.cumsum(mB.astype(jnp.int32), 1) - 1
    pos = jnp.where(mA, posA, jnp.where(mB, posB, LQP))
    iot = jnp.broadcast_to(jnp.arange(N, dtype=jnp.int32)[None], (B, N))
    bidx2 = jnp.arange(B)[:, None]
    out = jnp.zeros((B, LQP + 1), jnp.int32).at[bidx2, pos].set(
        iot, mode='drop')
    return out[:, :LQP]


RB = 32


def _idxwgt_body(q_ref, ref_ref, wb_ref, bb_ref, wa_ref, ba_ref,
                 wsx_ref, bsx_ref, wsy_ref, bsy_ref, idx_ref, wgt_ref):
    b = pl.program_id(0)
    q = q_ref[0]
    ob = jnp.dot(q, wb_ref[...], preferred_element_type=jnp.float32) + bb_ref[...]
    al = jnp.dot(q, wa_ref[...], preferred_element_type=jnp.float32) + ba_ref[...]
    spx = jnp.dot(q, wsx_ref[...], preferred_element_type=jnp.float32) + bsx_ref[...]
    spy = jnp.dot(q, wsy_ref[...], preferred_element_type=jnp.float32) + bsy_ref[...]
    refw = ref_ref[0]

    rcx, rcy = refw[:, 0:1], refw[:, 1:2]
    rw, rh, ang = refw[:, 3:4], refw[:, 4:5], refw[:, 6:7]
    ca = jnp.cos(ang)
    sa = jnp.sin(ang)
    kx, ky = _kxy(None)
    boff = b * (PG * PG * NH)

    idx_parts = []
    wgt_parts = []
    for h in range(NH):
        alh = al[:, h * NP_:(h + 1) * NP_]
        alh = alh - jnp.max(alh, axis=1, keepdims=True)
        ale = jnp.exp(alh)
        aw = ale / jnp.sum(ale, axis=1, keepdims=True)

        obx = ob[:, h * 4 + 0:h * 4 + 1]
        oby = ob[:, h * 4 + 1:h * 4 + 2]
        obw = ob[:, h * 4 + 2:h * 4 + 3]
        obh = ob[:, h * 4 + 3:h * 4 + 4]
        bx = rcx + obx / 8.0 * rw
        by = rcy + oby / 8.0 * rh
        bw = rw + obw / 8.0 * rw
        bh = rh + obh / 8.0 * rh
        sw = jnp.maximum(bw, 0.0)
        sh = jnp.maximum(bh, 0.0)
        fx = kx * sw
        fy = ky * sh
        gx = bx + fx * ca - fy * sa + spx[:, h * NP_:(h + 1) * NP_] / HSIZE
        gy = by + fx * sa + fy * ca + spy[:, h * NP_:(h + 1) * NP_] / HSIZE

        ix = gx * WGRID - 0.5
        iy = gy * HGRID - 0.5
        x0 = jnp.floor(ix)
        y0 = jnp.floor(iy)
        wx1 = ix - x0
        wx0 = 1.0 - wx1
        wy1 = iy - y0
        wy0 = 1.0 - wy1

        x0p = jnp.clip(x0 + 1.0, 0.0, PG - 1.0).astype(jnp.int32)
        y0p = jnp.clip(y0 + 1.0, 0.0, PG - 1.0).astype(jnp.int32)
        pidx = boff + (y0p * PG + x0p) * NH + h

        def cwgt(xi, yi, wx, wy):
            inb = ((xi >= 0.0) & (xi <= WGRID - 1.0)
                   & (yi >= 0.0) & (yi <= HGRID - 1.0))
            return aw * wx * wy * inb.astype(jnp.float32)

        idx_parts += [pidx]
        wgt_parts += [cwgt(x0, y0, wx0, wy0),
                      cwgt(x0 + 1.0, y0, wx1, wy0),
                      cwgt(x0, y0 + 1.0, wx0, wy1),
                      cwgt(x0 + 1.0, y0 + 1.0, wx1, wy1)]
    idx_ref[0] = jnp.concatenate(idx_parts, axis=1)
    wgt_ref[0] = jnp.concatenate(wgt_parts, axis=1)


def _idx_wgt(query, refp, p):
    grid = (B, LQP // RB)
    wsamp = p['samp_off_w'].reshape(NH * NP_, 2, D)
    bsamp = p['samp_off_b'].reshape(NH * NP_, 2)
    out = pl.pallas_call(
        _idxwgt_body,
        out_shape=(
            jax.ShapeDtypeStruct((B, LQP, NH * NP_), jnp.int32),
            jax.ShapeDtypeStruct((B, LQP, NH * NPTS), jnp.float32),
        ),
        grid=grid,
        in_specs=[
            pl.BlockSpec((1, RB, D), lambda b, i: (b, i, 0)),
            pl.BlockSpec((1, RB, 128), lambda b, i: (b, i, 0)),
            pl.BlockSpec((D, NH * 4), lambda b, i: (0, 0)),
            pl.BlockSpec((NH * 4,), lambda b, i: (0,)),
            pl.BlockSpec((D, NH * NP_), lambda b, i: (0, 0)),
            pl.BlockSpec((NH * NP_,), lambda b, i: (0,)),
            pl.BlockSpec((D, NH * NP_), lambda b, i: (0, 0)),
            pl.BlockSpec((NH * NP_,), lambda b, i: (0,)),
            pl.BlockSpec((D, NH * NP_), lambda b, i: (0, 0)),
            pl.BlockSpec((NH * NP_,), lambda b, i: (0,)),
        ],
        out_specs=(
            pl.BlockSpec((1, RB, NH * NP_), lambda b, i: (b, i, 0)),
            pl.BlockSpec((1, RB, NH * NPTS), lambda b, i: (b, i, 0)),
        ),
    )(query, refp,
      p['linear_box_w'].T, p['linear_box_b'],
      p['attn_w_w'].T, p['attn_w_b'],
      wsamp[:, 0, :].T, bsamp[:, 0],
      wsamp[:, 1, :].T, bsamp[:, 1])
    return out


def _sc_bag(vt, idxs, wgts):
    mesh = plsc.VectorSubcoreMesh(core_axis_name="c", subcore_axis_name="s")

    @functools.partial(
        pl.kernel, mesh=mesh,
        compiler_params=pltpu.CompilerParams(needs_layout_passes=False),
        out_type=jax.ShapeDtypeStruct((TOTQ, DH), jnp.float32),
        scratch_types=[
            pltpu.VMEM((2, CQ, NP_), jnp.int32),
            pltpu.VMEM((2 * CQ * NPTS,), jnp.float32),
            pltpu.VMEM((2, CQ * NP_, 128), jnp.float32),
            pltpu.VMEM((CQ, DH), jnp.float32),
            pltpu.SemaphoreType.DMA,
            pltpu.SemaphoreType.DMA,
        ],
    )
    def bag(vt_hbm, idx_hbm, wgt_hbm, out_hbm, idx_v, wgt_v, rows_v, out_v,
            gsem0, gsem1):
        wid = lax.axis_index("s") * 2 + lax.axis_index("c")
        base = wid * QPT
        sems = (gsem0, gsem1)

        def fire(s, off):
            pltpu.sync_copy(idx_hbm.at[pl.ds(off, CQ)], idx_v.at[s])
            pltpu.sync_copy(wgt_hbm.at[pl.ds(off * NPTS, CQ * NPTS)],
                            wgt_v.at[pl.ds(s * CQ * NPTS, CQ * NPTS)])
            for q in range(CQ):
                pltpu.async_copy(
                    vt_hbm.at[idx_v.at[s, q]],
                    rows_v.at[s, pl.ds(q * NP_, NP_)], sems[s])

        def wait_fired(s):
            for q in range(CQ):
                pltpu.make_async_copy(
                    vt_hbm.at[idx_v.at[s, q]],
                    rows_v.at[s, pl.ds(q * NP_, NP_)], sems[s]).wait()

        def compute(s, off):
            def qbody(q, _):
                rb = q * NP_
                qv = jnp.full((16,), s * CQ * NPTS + q * NPTS, jnp.int32)
                acc0 = jnp.zeros((16,), jnp.float32)
                acc1 = jnp.zeros((16,), jnp.float32)
                for p in range(NP_):
                    for c in range(4):
                        wv = plsc.load_gather(wgt_v, [qv + (c * NP_ + p)])
                        co = c * DH
                        acc0 = acc0 + wv * rows_v[s, rb + p, pl.ds(co, 16)]
                        acc1 = acc1 + wv * rows_v[s, rb + p,
                                                  pl.ds(co + 16, 16)]
                out_v[q, pl.ds(0, 16)] = acc0
                out_v[q, pl.ds(16, 16)] = acc1
                return 0

            lax.fori_loop(0, CQ, qbody, 0)
            pltpu.sync_copy(out_v, out_hbm.at[pl.ds(off, CQ)])

        fire(0, base)

        def pair(i, _):
            g0 = 2 * i
            fire(1, base + (g0 + 1) * CQ)
            wait_fired(0)
            compute(0, base + g0 * CQ)

            @pl.when(g0 + 2 < NCHUNK)
            def _():
                fire(0, base + (g0 + 2) * CQ)

            wait_fired(1)
            compute(1, base + (g0 + 1) * CQ)
            return 0

        lax.fori_loop(0, NCHUNK // 2, pair, 0)

    return bag(vt, idxs, wgts)


GR = 64
GPT = B * LQP // NTILES
GCH = GPT // GR


def _sc_select(src2, pos2, fidx):
    mesh = plsc.VectorSubcoreMesh(core_axis_name="c", subcore_axis_name="s")

    @functools.partial(
        pl.kernel, mesh=mesh,
        compiler_params=pltpu.CompilerParams(needs_layout_passes=False),
        out_type=(jax.ShapeDtypeStruct((B * LQP, D), jnp.float32),
                  jax.ShapeDtypeStruct((B * LQP, D), jnp.float32)),
        scratch_types=[
            pltpu.VMEM((GR,), jnp.int32),
            pltpu.VMEM((GR, D), jnp.float32),
            pltpu.VMEM((GR, D), jnp.float32),
            pltpu.SemaphoreType.DMA,
        ],
    )
    def sel(src_hbm, pos_hbm, fidx_hbm, osrc_hbm, opos_hbm,
            idx_v, bs_v, bp_v, sem):
        wid = lax.axis_index("s") * 2 + lax.axis_index("c")
        base = wid * GPT

        def chunk(c, _):
            off = base + c * GR
            pltpu.sync_copy(fidx_hbm.at[pl.ds(off, GR)], idx_v)
            cs = pltpu.async_copy(src_hbm.at[idx_v], bs_v, sem)
            cp = pltpu.async_copy(pos_hbm.at[idx_v], bp_v, sem)
            cs.wait()
            cp.wait()
            pltpu.sync_copy(bs_v, osrc_hbm.at[pl.ds(off, GR)])
            pltpu.sync_copy(bp_v, opos_hbm.at[pl.ds(off, GR)])
            return 0

        lax.fori_loop(0, GCH, chunk, 0)

    return sel(src2, pos2, fidx)


RB2 = 512


def _tail_body(sel_ref, bag_ref, wo_ref, bo_ref, g1_ref, b1_ref,
               w1_ref, bb1_ref, w2_ref, bb2_ref, g2_ref, b2_ref, o_ref):
    bagp = jnp.dot(bag_ref[0], wo_ref[...],
                   preferred_element_type=jnp.float32) + bo_ref[...]
    x = sel_ref[0] + bagp
    mu = x.mean(-1, keepdims=True)
    var = ((x - mu) ** 2).mean(-1, keepdims=True)
    x = (x - mu) / jnp.sqrt(var + 1e-5) * g1_ref[...] + b1_ref[...]
    h = jnp.maximum(
        jnp.dot(x, w1_ref[...], preferred_element_type=jnp.float32)
        + bb1_ref[...], 0.0)
    y = x + jnp.dot(h, w2_ref[...], preferred_element_type=jnp.float32) \
        + bb2_ref[...]
    mu = y.mean(-1, keepdims=True)
    var = ((y - mu) ** 2).mean(-1, keepdims=True)
    o_ref[0] = (y - mu) / jnp.sqrt(var + 1e-5) * g2_ref[...] + b2_ref[...]


def _tail(sel, bag, p):
    grid = (B * LQP // RB2,)
    sel = sel.reshape(B * LQP // RB2, RB2, D)
    bag = bag.reshape(B * LQP // RB2, RB2, D)
    vec = lambda: pl.BlockSpec((D,), lambda i: (0,))
    mat = lambda s: pl.BlockSpec(s, lambda i: (0, 0))
    out = pl.pallas_call(
        _tail_body,
        out_shape=jax.ShapeDtypeStruct((B * LQP // RB2, RB2, D), jnp.float32),
        grid=grid,
        in_specs=[
            pl.BlockSpec((1, RB2, D), lambda i: (i, 0, 0)),
            pl.BlockSpec((1, RB2, D), lambda i: (i, 0, 0)),
            mat((D, D)), vec(), vec(), vec(),
            mat((D, DFF)), pl.BlockSpec((DFF,), lambda i: (0,)),
            mat((DFF, D)), vec(), vec(), vec(),
        ],
        out_specs=pl.BlockSpec((1, RB2, D), lambda i: (i, 0, 0)),
    )(sel, bag, p['out_proj_w'].T, p['out_proj_b'],
      p['norm1_g'], p['norm1_b'],
      p['lin1_w'].T, p['lin1_b'], p['lin2_w'].T, p['lin2_b'],
      p['norm2_g'], p['norm2_b'])
    return out.reshape(B, LQP, D)


def kernel(src, pos, src_shape, src_start_idx, ref_windows, score_mask,
           params):
    p = params
    Bq = src.shape[0]
    indices_p = _select_indices(score_mask)
    indices = indices_p[:, :FG]
    select_ref = _gat(ref_windows, indices)
    fidx = indices_p + (jnp.arange(B, dtype=indices_p.dtype) * N)[:, None]
    sel_src_p, sel_pos_p = _sc_select(
        src.reshape(B * N, D), pos.reshape(B * N, D), fidx.reshape(B * LQP))
    select_src = sel_src_p.reshape(B, LQP, D)
    select_pos = sel_pos_p.reshape(B, LQP, D)
    query_src = select_src[:, :QN]
    query_pos = select_pos[:, :QN]
    q = query_src + query_pos
    q2 = _mha_small(q, q, query_src, p)
    query_src = _ln(query_src + q2, p['query_norm_g'], p['query_norm_b'])
    select_src = jnp.concatenate([query_src, select_src[:, QN:]], axis=1)

    v = (src.reshape(B * N, D) @ p['value_proj_w'].T + p['value_proj_b'])
    vg = jnp.pad(v.reshape(B, HGRID, WGRID, NH, DH),
                 ((0, 0), (1, 2), (1, 2), (0, 0), (0, 0)))
    vn = jnp.stack([vg[:, :-1, :-1], vg[:, :-1, 1:],
                    vg[:, 1:, :-1], vg[:, 1:, 1:]], axis=4)
    vt = vn.reshape(B * PG * PG * NH, 4 * DH)

    query = select_src + select_pos
    refp = jnp.pad(select_ref, ((0, 0), (0, LQP - FG), (0, 128 - 7)))
    idxs, wgts = _idx_wgt(query, refp, p)
    bag = _sc_bag(vt, idxs.reshape(TOTQ, NP_), wgts.reshape(TOTQ * NPTS))
    bag = bag.reshape(B, LQP, D)

    y = _tail(select_src, bag, p)[:, :FG]

    bidx = jnp.arange(Bq)[:, None]
    return src.at[bidx, indices].set(y)
